# baseline (speedup 1.0000x reference)
import jax
import jax.numpy as jnp
from jax import lax
from jax.experimental import pallas as pl
from jax.experimental.pallas import tpu as pltpu
from jax.experimental.pallas import tpu_sc as plsc

VOCAB = 1000000
EMB = 64
BATCH = 16384

_NC = 2
_NS = 16
_NW = _NC * _NS
_B_TOTAL = 2 * BATCH
_B_PER_W = _B_TOTAL // _NW
_CHUNK = 128
_ROWS_PER_BUF = 256
_N_BUFS = _B_PER_W // _CHUNK // 2

_VB = 1024
_NBLK = 245
_S = _NBLK * _VB
_LASTBLK = VOCAB // _VB

_M = 114688
_PAN_PER_W = _M // _NW // _CHUNK
_A = _S - _M
_TCBLK = _A // _VB
_TCOFF = _M // _VB


def _pack_pair(a, b):
    au = lax.bitcast_convert_type(a.astype(jnp.bfloat16), jnp.uint16).astype(jnp.uint32)
    bu = lax.bitcast_convert_type(b.astype(jnp.bfloat16), jnp.uint16).astype(jnp.uint32)
    return au | (bu << 16)


def _tc_repack_body(q0_ref, q1_ref, q2_ref, q3_ref, out_ref):
    p01 = _pack_pair(q0_ref[...], q1_ref[...])
    p23 = _pack_pair(q2_ref[...], q3_ref[...])
    out_ref[...] = lax.bitcast_convert_type(jnp.concatenate([p01.T, p23.T], axis=1), jnp.int32)


def _tc_repack(tablet):
    return pl.pallas_call(
        _tc_repack_body,
        grid=(_TCBLK,),
        in_specs=[
            pl.BlockSpec((EMB, _VB), lambda i: (0, i + _TCOFF)),
            pl.BlockSpec((EMB, _VB), lambda i: (0, i + _TCOFF + _NBLK)),
            pl.BlockSpec((EMB, _VB), lambda i: (0, i + _TCOFF + 2 * _NBLK)),
            pl.BlockSpec((EMB, _VB), lambda i: (0, jnp.minimum(i + _TCOFF + 3 * _NBLK, _LASTBLK))),
        ],
        out_specs=pl.BlockSpec((_VB, 2 * EMB), lambda i: (i, 0)),
        out_shape=jax.ShapeDtypeStruct((_A, 2 * EMB), jnp.int32),
    )(tablet, tablet, tablet, tablet)


def _round_bf16_bits(u):
    return (u + jnp.int32(0x8000)) & jnp.int32(-65536)


def _sc_repack_body(tablet_hbm, outb_hbm, pan0, pan1, outs, sem_in, sem_out):
    wid = lax.axis_index("s") * _NC + lax.axis_index("c")
    row0 = wid * _PAN_PER_W * _CHUNK

    def fetch(pan, p):
        col = row0 + p * _CHUNK
        return [
            pltpu.async_copy(
                tablet_hbm.at[:, pl.ds(pl.multiple_of(col + q * _S, _CHUNK), _CHUNK)],
                pan.at[q],
                sem_in,
            )
            for q in range(4)
        ]

    lanes = lax.iota(jnp.int32, 16)

    def compute(pan):
        def vbody(v, _):
            vv = jnp.zeros((16,), jnp.int32) + v
            for wb in range(4):
                wv = wb * 16 + lanes
                x0 = plsc.load_gather(pan, [jnp.zeros((16,), jnp.int32), wv, vv])
                x1 = plsc.load_gather(pan, [jnp.zeros((16,), jnp.int32) + 1, wv, vv])
                x2 = plsc.load_gather(pan, [jnp.zeros((16,), jnp.int32) + 2, wv, vv])
                x3 = plsc.load_gather(pan, [jnp.zeros((16,), jnp.int32) + 3, wv, vv])
                w01 = lax.shift_right_logical(_round_bf16_bits(x0), 16) | _round_bf16_bits(x1)
                w23 = lax.shift_right_logical(_round_bf16_bits(x2), 16) | _round_bf16_bits(x3)
                plsc.store_scatter(outs, [vv, wv], w01)
                plsc.store_scatter(outs, [vv, wv + 64], w23)
            return 0

        lax.fori_loop(0, _CHUNK, vbody, 0)

    def flush(p):
        pltpu.async_copy(outs, outb_hbm.at[pl.ds(row0 + p * _CHUNK, _CHUNK)], sem_out).wait()

    for c in fetch(pan0, 0):
        c.wait()
    for p in range(_PAN_PER_W):
        cur, nxt = (pan0, pan1) if p % 2 == 0 else (pan1, pan0)
        copies = fetch(nxt, p + 1) if p + 1 < _PAN_PER_W else []
        compute(cur)
        flush(p)
        for c in copies:
            c.wait()


def _sc_repack(tablet_i):
    mesh = plsc.VectorSubcoreMesh(core_axis_name="c", subcore_axis_name="s")
    return pl.kernel(
        _sc_repack_body,
        out_type=jax.ShapeDtypeStruct((_M, 2 * EMB), jnp.int32),
        mesh=mesh,
        scratch_types=[
            pltpu.VMEM((4, EMB, _CHUNK), jnp.int32),
            pltpu.VMEM((4, EMB, _CHUNK), jnp.int32),
            pltpu.VMEM((_CHUNK, 2 * EMB), jnp.int32),
            pltpu.SemaphoreType.DMA,
            pltpu.SemaphoreType.DMA,
        ],
        compiler_params=pltpu.CompilerParams(
            use_tc_tiling_on_sc=True, needs_layout_passes=False),
    )(tablet_i)


def _sc_gather_body(pa_hbm, pb_hbm, ia_hbm, ib_hbm, oa_hbm, ob_hbm,
                    ia_v, ib_v, rows_a, rows_b, sem):
    wid = lax.axis_index("s") * _NC + lax.axis_index("c")
    base = wid * _B_PER_W
    nidx = _B_PER_W // _CHUNK
    pltpu.sync_copy(ia_hbm.at[pl.ds(wid * nidx, nidx)], ia_v)
    pltpu.sync_copy(ib_hbm.at[pl.ds(wid * nidx, nidx)], ib_v)
    for c in range(4):
        copies = []
        for j in range(2):
            k = c * 2 + j
            copies.append(pltpu.async_copy(
                pa_hbm.at[ia_v.at[k]], rows_a.at[pl.ds(j * _CHUNK, _CHUNK)], sem))
            copies.append(pltpu.async_copy(
                pb_hbm.at[ib_v.at[k]], rows_b.at[pl.ds(j * _CHUNK, _CHUNK)], sem))
        for cp in copies:
            cp.wait()
        off = base + c * 2 * _CHUNK
        pltpu.sync_copy(rows_a, oa_hbm.at[pl.ds(off, 2 * _CHUNK)])
        pltpu.sync_copy(rows_b, ob_hbm.at[pl.ds(off, 2 * _CHUNK)])


def _sc_gather(pa, pb, ia2d, ib2d):
    mesh = plsc.VectorSubcoreMesh(core_axis_name="c", subcore_axis_name="s")
    return pl.kernel(
        _sc_gather_body,
        out_type=(
            jax.ShapeDtypeStruct((_B_TOTAL, 2 * EMB), jnp.int32),
            jax.ShapeDtypeStruct((_B_TOTAL, 2 * EMB), jnp.int32),
        ),
        mesh=mesh,
        scratch_types=[
            pltpu.VMEM((_B_PER_W // _CHUNK, _CHUNK), jnp.int32),
            pltpu.VMEM((_B_PER_W // _CHUNK, _CHUNK), jnp.int32),
            pltpu.VMEM((2 * _CHUNK, 2 * EMB), jnp.int32),
            pltpu.VMEM((2 * _CHUNK, 2 * EMB), jnp.int32),
            pltpu.SemaphoreType.DMA,
        ],
        compiler_params=pltpu.CompilerParams(use_tc_tiling_on_sc=True),
    )(pa, pb, ia2d, ib2d)


_BM = 2048


def _unpack_select(xa, xb, sb, q):
    x = lax.bitcast_convert_type(jnp.where(sb == 1, xb, xa), jnp.uint32)
    xh = jnp.where(q >= 2, x[:, EMB:], x[:, :EMB])
    lo_f = lax.bitcast_convert_type(xh << 16, jnp.float32)
    hi_f = lax.bitcast_convert_type(xh & jnp.uint32(0xFFFF0000), jnp.float32)
    return jnp.where((q & 1) == 1, hi_f, lo_f)


def _mlp_body(xa1_ref, xb1_ref, xa2_ref, xb2_ref, s1_ref, q1_ref, s2_ref, q2_ref,
              w1_ref, b1_ref, w2t_ref, b2_ref, out_ref):
    w1 = w1_ref[...]
    h_emb = _unpack_select(xa1_ref[...], xb1_ref[...], s1_ref[...], q1_ref[...])
    t_emb = _unpack_select(xa2_ref[...], xb2_ref[...], s2_ref[...], q2_ref[...])
    h = jnp.dot(h_emb, w1[:EMB], preferred_element_type=jnp.float32)
    h = h + jnp.dot(t_emb, w1[EMB:], preferred_element_type=jnp.float32)
    h = jnp.maximum(h + b1_ref[...], 0.0)
    out = jnp.sum(h * w2t_ref[...], axis=1, keepdims=True) + b2_ref[...]
    out_ref[...] = out


def _tc_mlp(ea, eb, selb, quarter, W1, b1, W2, b2):
    nblk = BATCH // _BM
    bs = lambda off: pl.BlockSpec((_BM, 2 * EMB), lambda i, off=off: (i + off, 0))
    ss = lambda off: pl.BlockSpec((_BM, 1), lambda i, off=off: (i + off, 0))
    return pl.pallas_call(
        _mlp_body,
        grid=(nblk,),
        in_specs=[
            bs(0), bs(0), bs(nblk), bs(nblk),
            ss(0), ss(0), ss(nblk), ss(nblk),
            pl.BlockSpec((2 * EMB, EMB), lambda i: (0, 0)),
            pl.BlockSpec((1, EMB), lambda i: (0, 0)),
            pl.BlockSpec((1, EMB), lambda i: (0, 0)),
            pl.BlockSpec((1, 1), lambda i: (0, 0)),
        ],
        out_specs=pl.BlockSpec((_BM, 1), lambda i: (i, 0)),
        out_shape=jax.ShapeDtypeStruct((BATCH, 1), jnp.float32),
    )(ea, eb, ea, eb, selb, quarter, selb, quarter,
      W1, b1.reshape(1, EMB), W2.reshape(1, EMB), b2.reshape(1, 1))


def kernel(head, tail, table, W1, b1, W2, b2):
    idx = jnp.concatenate([head, tail]).astype(jnp.int32)
    q = idx // _S
    g = idx - q * _S
    selb = (g < _M).astype(jnp.int32)
    gb = jnp.minimum(g, _M - 1)
    ga = jnp.clip(g - _M, 0, _A - 1)
    tablet = table.T
    packed_a = _tc_repack(tablet)
    packed_b = _sc_repack(lax.bitcast_convert_type(table, jnp.int32).T)
    ea, eb = _sc_gather(
        packed_a, packed_b,
        ga.reshape(_B_TOTAL // _CHUNK, _CHUNK),
        gb.reshape(_B_TOTAL // _CHUNK, _CHUNK),
    )
    return _tc_mlp(ea, eb, selb.reshape(_B_TOTAL, 1), q.reshape(_B_TOTAL, 1),
                   W1, b1, W2, b2)


# speedup vs baseline: 4.5801x; 4.5801x over previous
import jax
import jax.numpy as jnp
from jax import lax
from jax.experimental import pallas as pl
from jax.experimental.pallas import tpu as pltpu
from jax.experimental.pallas import tpu_sc as plsc

VOCAB = 1000000
EMB = 64
BATCH = 16384

_NC = 2
_NS = 16
_NW = _NC * _NS
_B_TOTAL = 2 * BATCH
_B_PER_W = _B_TOTAL // _NW
_CHUNK = 128
_ROWS_PER_BUF = 256
_N_BUFS = _B_PER_W // _ROWS_PER_BUF

_VB = 1024
_NBLK = 245
_S = _NBLK * _VB
_LASTBLK = VOCAB // _VB


def _pack_pair(a, b):
    au = lax.bitcast_convert_type(a.astype(jnp.bfloat16), jnp.uint16).astype(jnp.uint32)
    bu = lax.bitcast_convert_type(b.astype(jnp.bfloat16), jnp.uint16).astype(jnp.uint32)
    return au | (bu << 16)


def _repack_body(q0_ref, q1_ref, q2_ref, q3_ref, out_ref):
    p01 = _pack_pair(q0_ref[...], q1_ref[...])
    p23 = _pack_pair(q2_ref[...], q3_ref[...])
    out_ref[...] = lax.bitcast_convert_type(
        jnp.concatenate([p01.T, p23.T], axis=1), jnp.int32)


def _tc_repack(tablet):
    return pl.pallas_call(
        _repack_body,
        grid=(_NBLK,),
        in_specs=[
            pl.BlockSpec((EMB, _VB), lambda i: (0, i)),
            pl.BlockSpec((EMB, _VB), lambda i: (0, i + _NBLK)),
            pl.BlockSpec((EMB, _VB), lambda i: (0, i + 2 * _NBLK)),
            pl.BlockSpec((EMB, _VB), lambda i: (0, jnp.minimum(i + 3 * _NBLK, _LASTBLK))),
        ],
        out_specs=pl.BlockSpec((_VB, 2 * EMB), lambda i: (i, 0)),
        out_shape=jax.ShapeDtypeStruct((_S, 2 * EMB), jnp.int32),
    )(tablet, tablet, tablet, tablet)


def _sc_gather_body(packed_hbm, idx_hbm, out_hbm, idx_v, rows_v, sem):
    wid = lax.axis_index("s") * _NC + lax.axis_index("c")
    base = wid * _B_PER_W
    pltpu.sync_copy(idx_hbm.at[pl.ds(wid * (_B_PER_W // _CHUNK), _B_PER_W // _CHUNK)], idx_v)
    for c in range(_N_BUFS):
        copies = [
            pltpu.async_copy(
                packed_hbm.at[idx_v.at[c * (_ROWS_PER_BUF // _CHUNK) + j]],
                rows_v.at[pl.ds(j * _CHUNK, _CHUNK)],
                sem,
            )
            for j in range(_ROWS_PER_BUF // _CHUNK)
        ]
        for cp in copies:
            cp.wait()
        pltpu.sync_copy(rows_v, out_hbm.at[pl.ds(base + c * _ROWS_PER_BUF, _ROWS_PER_BUF)])


def _sc_gather(packed, idx2d):
    mesh = plsc.VectorSubcoreMesh(core_axis_name="c", subcore_axis_name="s")
    return pl.kernel(
        _sc_gather_body,
        out_type=jax.ShapeDtypeStruct((_B_TOTAL, 2 * EMB), jnp.int32),
        mesh=mesh,
        scratch_types=[
            pltpu.VMEM((_B_PER_W // _CHUNK, _CHUNK), jnp.int32),
            pltpu.VMEM((_ROWS_PER_BUF, 2 * EMB), jnp.int32),
            pltpu.SemaphoreType.DMA,
        ],
        compiler_params=pltpu.CompilerParams(use_tc_tiling_on_sc=True),
    )(packed, idx2d)


_BM = 2048


def _unpack_select(x, q):
    xu = lax.bitcast_convert_type(x, jnp.uint32)
    xh = jnp.where(q >= 2, xu[:, EMB:], xu[:, :EMB])
    lo_f = lax.bitcast_convert_type(xh << 16, jnp.float32)
    hi_f = lax.bitcast_convert_type(xh & jnp.uint32(0xFFFF0000), jnp.float32)
    return jnp.where((q & 1) == 1, hi_f, lo_f)


def _mlp_body(x1_ref, x2_ref, q1_ref, q2_ref, w1_ref, b1_ref, w2t_ref, b2_ref, out_ref):
    w1 = w1_ref[...]
    h_emb = _unpack_select(x1_ref[...], q1_ref[...])
    t_emb = _unpack_select(x2_ref[...], q2_ref[...])
    h = jnp.dot(h_emb, w1[:EMB], preferred_element_type=jnp.float32)
    h = h + jnp.dot(t_emb, w1[EMB:], preferred_element_type=jnp.float32)
    h = jnp.maximum(h + b1_ref[...], 0.0)
    out = jnp.sum(h * w2t_ref[...], axis=1, keepdims=True) + b2_ref[...]
    out_ref[...] = out


def _tc_mlp(embg, quarter, W1, b1, W2, b2):
    nblk = BATCH // _BM
    return pl.pallas_call(
        _mlp_body,
        grid=(nblk,),
        in_specs=[
            pl.BlockSpec((_BM, 2 * EMB), lambda i: (i, 0)),
            pl.BlockSpec((_BM, 2 * EMB), lambda i: (i + nblk, 0)),
            pl.BlockSpec((_BM, 1), lambda i: (i, 0)),
            pl.BlockSpec((_BM, 1), lambda i: (i + nblk, 0)),
            pl.BlockSpec((2 * EMB, EMB), lambda i: (0, 0)),
            pl.BlockSpec((1, EMB), lambda i: (0, 0)),
            pl.BlockSpec((1, EMB), lambda i: (0, 0)),
            pl.BlockSpec((1, 1), lambda i: (0, 0)),
        ],
        out_specs=pl.BlockSpec((_BM, 1), lambda i: (i, 0)),
        out_shape=jax.ShapeDtypeStruct((BATCH, 1), jnp.float32),
    )(embg, embg, quarter, quarter, W1, b1.reshape(1, EMB), W2.reshape(1, EMB), b2.reshape(1, 1))


def kernel(head, tail, table, W1, b1, W2, b2):
    idx = jnp.concatenate([head, tail]).astype(jnp.int32)
    q = idx // _S
    g = idx - q * _S
    g2d = g.reshape(_B_TOTAL // _CHUNK, _CHUNK)
    packed = _tc_repack(table.T)
    embg = _sc_gather(packed, g2d)
    return _tc_mlp(embg, q.reshape(_B_TOTAL, 1), W1, b1, W2, b2)


# speedup vs baseline: 4.6287x; 1.0106x over previous
import jax
import jax.numpy as jnp
from jax import lax
from jax.experimental import pallas as pl
from jax.experimental.pallas import tpu as pltpu
from jax.experimental.pallas import tpu_sc as plsc

VOCAB = 1000000
EMB = 64
BATCH = 16384

_NC = 2
_NS = 16
_NW = _NC * _NS
_B_TOTAL = 2 * BATCH
_B_PER_W = _B_TOTAL // _NW
_CHUNK = 128
_ROWS_PER_BUF = 512
_N_BUFS = _B_PER_W // _ROWS_PER_BUF

_VB = 1024
_NBLK = 245
_S = _NBLK * _VB
_LASTBLK = VOCAB // _VB


def _pack_pair(a, b):
    au = lax.bitcast_convert_type(a.astype(jnp.bfloat16), jnp.uint16).astype(jnp.uint32)
    bu = lax.bitcast_convert_type(b.astype(jnp.bfloat16), jnp.uint16).astype(jnp.uint32)
    return au | (bu << 16)


def _repack_body(q0_ref, q1_ref, q2_ref, q3_ref, out_ref):
    p01 = _pack_pair(q0_ref[...], q1_ref[...])
    p23 = _pack_pair(q2_ref[...], q3_ref[...])
    out_ref[...] = lax.bitcast_convert_type(
        jnp.concatenate([p01.T, p23.T], axis=1), jnp.int32)


def _tc_repack(tablet):
    return pl.pallas_call(
        _repack_body,
        grid=(_NBLK,),
        in_specs=[
            pl.BlockSpec((EMB, _VB), lambda i: (0, i)),
            pl.BlockSpec((EMB, _VB), lambda i: (0, i + _NBLK)),
            pl.BlockSpec((EMB, _VB), lambda i: (0, i + 2 * _NBLK)),
            pl.BlockSpec((EMB, _VB), lambda i: (0, jnp.minimum(i + 3 * _NBLK, _LASTBLK))),
        ],
        out_specs=pl.BlockSpec((_VB, 2 * EMB), lambda i: (i, 0)),
        out_shape=jax.ShapeDtypeStruct((_S, 2 * EMB), jnp.int32),
    )(tablet, tablet, tablet, tablet)


def _sc_gather_body(packed_hbm, idx_hbm, out_hbm, idx_v, rows_v, sem):
    wid = lax.axis_index("s") * _NC + lax.axis_index("c")
    base = wid * _B_PER_W
    pltpu.sync_copy(idx_hbm.at[pl.ds(wid * (_B_PER_W // _CHUNK), _B_PER_W // _CHUNK)], idx_v)
    for c in range(_N_BUFS):
        copies = [
            pltpu.async_copy(
                packed_hbm.at[idx_v.at[c * (_ROWS_PER_BUF // _CHUNK) + j]],
                rows_v.at[pl.ds(j * _CHUNK, _CHUNK)],
                sem,
            )
            for j in range(_ROWS_PER_BUF // _CHUNK)
        ]
        for cp in copies:
            cp.wait()
        pltpu.sync_copy(rows_v, out_hbm.at[pl.ds(base + c * _ROWS_PER_BUF, _ROWS_PER_BUF)])


def _sc_gather(packed, idx2d):
    mesh = plsc.VectorSubcoreMesh(core_axis_name="c", subcore_axis_name="s")
    return pl.kernel(
        _sc_gather_body,
        out_type=jax.ShapeDtypeStruct((_B_TOTAL, 2 * EMB), jnp.int32),
        mesh=mesh,
        scratch_types=[
            pltpu.VMEM((_B_PER_W // _CHUNK, _CHUNK), jnp.int32),
            pltpu.VMEM((_ROWS_PER_BUF, 2 * EMB), jnp.int32),
            pltpu.SemaphoreType.DMA,
        ],
        compiler_params=pltpu.CompilerParams(use_tc_tiling_on_sc=True),
    )(packed, idx2d)


_BM = 4096


def _unpack_select(x, q):
    xu = lax.bitcast_convert_type(x, jnp.uint32)
    xh = jnp.where(q >= 2, xu[:, EMB:], xu[:, :EMB])
    lo_f = lax.bitcast_convert_type(xh << 16, jnp.float32)
    hi_f = lax.bitcast_convert_type(xh & jnp.uint32(0xFFFF0000), jnp.float32)
    return jnp.where((q & 1) == 1, hi_f, lo_f)


def _mlp_body(x1_ref, x2_ref, q1_ref, q2_ref, w1_ref, b1_ref, w2t_ref, b2_ref, out_ref):
    w1 = w1_ref[...]
    h_emb = _unpack_select(x1_ref[...], q1_ref[...])
    t_emb = _unpack_select(x2_ref[...], q2_ref[...])
    h = jnp.dot(h_emb, w1[:EMB], preferred_element_type=jnp.float32)
    h = h + jnp.dot(t_emb, w1[EMB:], preferred_element_type=jnp.float32)
    h = jnp.maximum(h + b1_ref[...], 0.0)
    out = jnp.sum(h * w2t_ref[...], axis=1, keepdims=True) + b2_ref[...]
    out_ref[...] = out


def _tc_mlp(embg, quarter, W1, b1, W2, b2):
    nblk = BATCH // _BM
    return pl.pallas_call(
        _mlp_body,
        grid=(nblk,),
        in_specs=[
            pl.BlockSpec((_BM, 2 * EMB), lambda i: (i, 0)),
            pl.BlockSpec((_BM, 2 * EMB), lambda i: (i + nblk, 0)),
            pl.BlockSpec((_BM, 1), lambda i: (i, 0)),
            pl.BlockSpec((_BM, 1), lambda i: (i + nblk, 0)),
            pl.BlockSpec((2 * EMB, EMB), lambda i: (0, 0)),
            pl.BlockSpec((1, EMB), lambda i: (0, 0)),
            pl.BlockSpec((1, EMB), lambda i: (0, 0)),
            pl.BlockSpec((1, 1), lambda i: (0, 0)),
        ],
        out_specs=pl.BlockSpec((_BM, 1), lambda i: (i, 0)),
        out_shape=jax.ShapeDtypeStruct((BATCH, 1), jnp.float32),
    )(embg, embg, quarter, quarter, W1, b1.reshape(1, EMB), W2.reshape(1, EMB), b2.reshape(1, 1))


def kernel(head, tail, table, W1, b1, W2, b2):
    idx = jnp.concatenate([head, tail]).astype(jnp.int32)
    q = idx // _S
    g = idx - q * _S
    g2d = g.reshape(_B_TOTAL // _CHUNK, _CHUNK)
    packed = _tc_repack(table.T)
    embg = _sc_gather(packed, g2d)
    return _tc_mlp(embg, q.reshape(_B_TOTAL, 1), W1, b1, W2, b2)


# speedup vs baseline: 5.5770x; 1.2049x over previous
import jax
import jax.numpy as jnp
from jax import lax
from jax.experimental import pallas as pl
from jax.experimental.pallas import tpu as pltpu
from jax.experimental.pallas import tpu_sc as plsc

VOCAB = 1000000
EMB = 64
BATCH = 16384

_NC = 2
_NS = 16
_NW = _NC * _NS
_B_TOTAL = 2 * BATCH
_B_PER_W = _B_TOTAL // _NW
_CHUNK = 128
_ROWS_PER_BUF = 512
_N_BUFS = _B_PER_W // _ROWS_PER_BUF

_VB = 2048
_NBLK = 123
_S = _NBLK * _VB
_LASTBLK = VOCAB // _VB


def _pack_pair(a, b):
    au = lax.bitcast_convert_type(a.astype(jnp.bfloat16), jnp.uint16).astype(jnp.uint32)
    bu = lax.bitcast_convert_type(b.astype(jnp.bfloat16), jnp.uint16).astype(jnp.uint32)
    return au | (bu << 16)


def _repack_body(q0_ref, q1_ref, q2_ref, q3_ref, out_ref):
    p01 = _pack_pair(q0_ref[...], q1_ref[...])
    p23 = _pack_pair(q2_ref[...], q3_ref[...])
    out_ref[...] = lax.bitcast_convert_type(
        jnp.concatenate([p01.T, p23.T], axis=1), jnp.int32)


def _tc_repack(tablet):
    return pl.pallas_call(
        _repack_body,
        grid=(_NBLK,),
        in_specs=[
            pl.BlockSpec((EMB, _VB), lambda i: (0, i)),
            pl.BlockSpec((EMB, _VB), lambda i: (0, i + _NBLK)),
            pl.BlockSpec((EMB, _VB), lambda i: (0, i + 2 * _NBLK)),
            pl.BlockSpec((EMB, _VB), lambda i: (0, jnp.minimum(i + 3 * _NBLK, _LASTBLK))),
        ],
        out_specs=pl.BlockSpec((_VB, 2 * EMB), lambda i: (i, 0)),
        out_shape=jax.ShapeDtypeStruct((_S, 2 * EMB), jnp.int32),
    )(tablet, tablet, tablet, tablet)


def _sc_gather_body(packed_hbm, idx_hbm, out_hbm, idx_v, rows_v, sem):
    wid = lax.axis_index("s") * _NC + lax.axis_index("c")
    base = wid * _B_PER_W
    pltpu.sync_copy(idx_hbm.at[pl.ds(wid * (_B_PER_W // _CHUNK), _B_PER_W // _CHUNK)], idx_v)
    for c in range(_N_BUFS):
        copies = [
            pltpu.async_copy(
                packed_hbm.at[idx_v.at[c * (_ROWS_PER_BUF // _CHUNK) + j]],
                rows_v.at[pl.ds(j * _CHUNK, _CHUNK)],
                sem,
            )
            for j in range(_ROWS_PER_BUF // _CHUNK)
        ]
        for cp in copies:
            cp.wait()
        pltpu.sync_copy(rows_v, out_hbm.at[pl.ds(base + c * _ROWS_PER_BUF, _ROWS_PER_BUF)])


def _sc_gather(packed, idx2d):
    mesh = plsc.VectorSubcoreMesh(core_axis_name="c", subcore_axis_name="s")
    return pl.kernel(
        _sc_gather_body,
        out_type=jax.ShapeDtypeStruct((_B_TOTAL, 2 * EMB), jnp.int32),
        mesh=mesh,
        scratch_types=[
            pltpu.VMEM((_B_PER_W // _CHUNK, _CHUNK), jnp.int32),
            pltpu.VMEM((_ROWS_PER_BUF, 2 * EMB), jnp.int32),
            pltpu.SemaphoreType.DMA,
        ],
        compiler_params=pltpu.CompilerParams(use_tc_tiling_on_sc=True),
    )(packed, idx2d)


_BM = 4096


def _unpack_select(x, q):
    xu = lax.bitcast_convert_type(x, jnp.uint32)
    xh = jnp.where(q >= 2, xu[:, EMB:], xu[:, :EMB])
    lo_f = lax.bitcast_convert_type(xh << 16, jnp.float32)
    hi_f = lax.bitcast_convert_type(xh & jnp.uint32(0xFFFF0000), jnp.float32)
    return jnp.where((q & 1) == 1, hi_f, lo_f)


def _mlp_body(x1_ref, x2_ref, q1_ref, q2_ref, w1_ref, b1_ref, w2t_ref, b2_ref, out_ref):
    w1 = w1_ref[...]
    h_emb = _unpack_select(x1_ref[...], q1_ref[...])
    t_emb = _unpack_select(x2_ref[...], q2_ref[...])
    h = jnp.dot(h_emb, w1[:EMB], preferred_element_type=jnp.float32)
    h = h + jnp.dot(t_emb, w1[EMB:], preferred_element_type=jnp.float32)
    h = jnp.maximum(h + b1_ref[...], 0.0)
    out = jnp.sum(h * w2t_ref[...], axis=1, keepdims=True) + b2_ref[...]
    out_ref[...] = out


def _tc_mlp(embg, quarter, W1, b1, W2, b2):
    nblk = BATCH // _BM
    return pl.pallas_call(
        _mlp_body,
        grid=(nblk,),
        in_specs=[
            pl.BlockSpec((_BM, 2 * EMB), lambda i: (i, 0)),
            pl.BlockSpec((_BM, 2 * EMB), lambda i: (i + nblk, 0)),
            pl.BlockSpec((_BM, 1), lambda i: (i, 0)),
            pl.BlockSpec((_BM, 1), lambda i: (i + nblk, 0)),
            pl.BlockSpec((2 * EMB, EMB), lambda i: (0, 0)),
            pl.BlockSpec((1, EMB), lambda i: (0, 0)),
            pl.BlockSpec((1, EMB), lambda i: (0, 0)),
            pl.BlockSpec((1, 1), lambda i: (0, 0)),
        ],
        out_specs=pl.BlockSpec((_BM, 1), lambda i: (i, 0)),
        out_shape=jax.ShapeDtypeStruct((BATCH, 1), jnp.float32),
    )(embg, embg, quarter, quarter, W1, b1.reshape(1, EMB), W2.reshape(1, EMB), b2.reshape(1, 1))


def kernel(head, tail, table, W1, b1, W2, b2):
    idx = jnp.concatenate([head, tail]).astype(jnp.int32)
    q = idx // _S
    g = idx - q * _S
    g2d = g.reshape(_B_TOTAL // _CHUNK, _CHUNK)
    packed = _tc_repack(table.T)
    embg = _sc_gather(packed, g2d)
    return _tc_mlp(embg, q.reshape(_B_TOTAL, 1), W1, b1, W2, b2)


# speedup vs baseline: 6.3854x; 1.1449x over previous
import jax
import jax.numpy as jnp
from jax import lax
from jax.experimental import pallas as pl
from jax.experimental.pallas import tpu as pltpu
from jax.experimental.pallas import tpu_sc as plsc

VOCAB = 1000000
EMB = 64
BATCH = 16384

_NC = 2
_NS = 16
_NW = _NC * _NS
_B_TOTAL = 2 * BATCH
_B_PER_W = _B_TOTAL // _NW
_CHUNK = 128
_ROWS_PER_BUF = 512
_N_BUFS = _B_PER_W // _ROWS_PER_BUF

_VB = 4096
_NBLK = 62
_S = _NBLK * _VB
_LASTBLK = VOCAB // _VB


def _pack_pair(a, b):
    au = lax.bitcast_convert_type(a.astype(jnp.bfloat16), jnp.uint16).astype(jnp.uint32)
    bu = lax.bitcast_convert_type(b.astype(jnp.bfloat16), jnp.uint16).astype(jnp.uint32)
    return au | (bu << 16)


def _repack_body(q0_ref, q1_ref, q2_ref, q3_ref, out_ref):
    p01 = _pack_pair(q0_ref[...], q1_ref[...])
    p23 = _pack_pair(q2_ref[...], q3_ref[...])
    out_ref[...] = lax.bitcast_convert_type(
        jnp.concatenate([p01.T, p23.T], axis=1), jnp.int32)


def _tc_repack(tablet):
    return pl.pallas_call(
        _repack_body,
        grid=(_NBLK,),
        in_specs=[
            pl.BlockSpec((EMB, _VB), lambda i: (0, i)),
            pl.BlockSpec((EMB, _VB), lambda i: (0, i + _NBLK)),
            pl.BlockSpec((EMB, _VB), lambda i: (0, i + 2 * _NBLK)),
            pl.BlockSpec((EMB, _VB), lambda i: (0, jnp.minimum(i + 3 * _NBLK, _LASTBLK))),
        ],
        out_specs=pl.BlockSpec((_VB, 2 * EMB), lambda i: (i, 0)),
        out_shape=jax.ShapeDtypeStruct((_S, 2 * EMB), jnp.int32),
    )(tablet, tablet, tablet, tablet)


def _sc_gather_body(packed_hbm, idx_hbm, out_hbm, idx_v, rows_v, sem):
    wid = lax.axis_index("s") * _NC + lax.axis_index("c")
    base = wid * _B_PER_W
    pltpu.sync_copy(idx_hbm.at[pl.ds(wid * (_B_PER_W // _CHUNK), _B_PER_W // _CHUNK)], idx_v)
    for c in range(_N_BUFS):
        copies = [
            pltpu.async_copy(
                packed_hbm.at[idx_v.at[c * (_ROWS_PER_BUF // _CHUNK) + j]],
                rows_v.at[pl.ds(j * _CHUNK, _CHUNK)],
                sem,
            )
            for j in range(_ROWS_PER_BUF // _CHUNK)
        ]
        for cp in copies:
            cp.wait()
        pltpu.sync_copy(rows_v, out_hbm.at[pl.ds(base + c * _ROWS_PER_BUF, _ROWS_PER_BUF)])


def _sc_gather(packed, idx2d):
    mesh = plsc.VectorSubcoreMesh(core_axis_name="c", subcore_axis_name="s")
    return pl.kernel(
        _sc_gather_body,
        out_type=jax.ShapeDtypeStruct((_B_TOTAL, 2 * EMB), jnp.int32),
        mesh=mesh,
        scratch_types=[
            pltpu.VMEM((_B_PER_W // _CHUNK, _CHUNK), jnp.int32),
            pltpu.VMEM((_ROWS_PER_BUF, 2 * EMB), jnp.int32),
            pltpu.SemaphoreType.DMA,
        ],
        compiler_params=pltpu.CompilerParams(use_tc_tiling_on_sc=True),
    )(packed, idx2d)


_BM = 4096


def _unpack_select(x, q):
    xu = lax.bitcast_convert_type(x, jnp.uint32)
    xh = jnp.where(q >= 2, xu[:, EMB:], xu[:, :EMB])
    lo_f = lax.bitcast_convert_type(xh << 16, jnp.float32)
    hi_f = lax.bitcast_convert_type(xh & jnp.uint32(0xFFFF0000), jnp.float32)
    return jnp.where((q & 1) == 1, hi_f, lo_f)


def _mlp_body(x1_ref, x2_ref, q1_ref, q2_ref, w1_ref, b1_ref, w2t_ref, b2_ref, out_ref):
    w1 = w1_ref[...]
    h_emb = _unpack_select(x1_ref[...], q1_ref[...])
    t_emb = _unpack_select(x2_ref[...], q2_ref[...])
    h = jnp.dot(h_emb, w1[:EMB], preferred_element_type=jnp.float32)
    h = h + jnp.dot(t_emb, w1[EMB:], preferred_element_type=jnp.float32)
    h = jnp.maximum(h + b1_ref[...], 0.0)
    out = jnp.sum(h * w2t_ref[...], axis=1, keepdims=True) + b2_ref[...]
    out_ref[...] = out


def _tc_mlp(embg, quarter, W1, b1, W2, b2):
    nblk = BATCH // _BM
    return pl.pallas_call(
        _mlp_body,
        grid=(nblk,),
        in_specs=[
            pl.BlockSpec((_BM, 2 * EMB), lambda i: (i, 0)),
            pl.BlockSpec((_BM, 2 * EMB), lambda i: (i + nblk, 0)),
            pl.BlockSpec((_BM, 1), lambda i: (i, 0)),
            pl.BlockSpec((_BM, 1), lambda i: (i + nblk, 0)),
            pl.BlockSpec((2 * EMB, EMB), lambda i: (0, 0)),
            pl.BlockSpec((1, EMB), lambda i: (0, 0)),
            pl.BlockSpec((1, EMB), lambda i: (0, 0)),
            pl.BlockSpec((1, 1), lambda i: (0, 0)),
        ],
        out_specs=pl.BlockSpec((_BM, 1), lambda i: (i, 0)),
        out_shape=jax.ShapeDtypeStruct((BATCH, 1), jnp.float32),
    )(embg, embg, quarter, quarter, W1, b1.reshape(1, EMB), W2.reshape(1, EMB), b2.reshape(1, 1))


def kernel(head, tail, table, W1, b1, W2, b2):
    idx = jnp.concatenate([head, tail]).astype(jnp.int32)
    q = idx // _S
    g = idx - q * _S
    g2d = g.reshape(_B_TOTAL // _CHUNK, _CHUNK)
    packed = _tc_repack(table.T)
    embg = _sc_gather(packed, g2d)
    return _tc_mlp(embg, q.reshape(_B_TOTAL, 1), W1, b1, W2, b2)


# speedup vs baseline: 6.8387x; 1.0710x over previous
import jax
import jax.numpy as jnp
from jax import lax
from jax.experimental import pallas as pl
from jax.experimental.pallas import tpu as pltpu
from jax.experimental.pallas import tpu_sc as plsc

VOCAB = 1000000
EMB = 64
BATCH = 16384

_NC = 2
_NS = 16
_NW = _NC * _NS
_B_TOTAL = 2 * BATCH
_B_PER_W = _B_TOTAL // _NW
_CHUNK = 128
_ROWS_PER_BUF = 512
_N_BUFS = _B_PER_W // _ROWS_PER_BUF

_VB = 8192
_NBLK = 31
_S = _NBLK * _VB
_LASTBLK = VOCAB // _VB


def _pack_pair(a, b):
    au = lax.bitcast_convert_type(a.astype(jnp.bfloat16), jnp.uint16).astype(jnp.uint32)
    bu = lax.bitcast_convert_type(b.astype(jnp.bfloat16), jnp.uint16).astype(jnp.uint32)
    return au | (bu << 16)


def _repack_body(q0_ref, q1_ref, q2_ref, q3_ref, out_ref):
    p01 = _pack_pair(q0_ref[...], q1_ref[...])
    p23 = _pack_pair(q2_ref[...], q3_ref[...])
    out_ref[...] = lax.bitcast_convert_type(
        jnp.concatenate([p01.T, p23.T], axis=1), jnp.int32)


def _tc_repack(tablet):
    return pl.pallas_call(
        _repack_body,
        grid=(_NBLK,),
        in_specs=[
            pl.BlockSpec((EMB, _VB), lambda i: (0, i)),
            pl.BlockSpec((EMB, _VB), lambda i: (0, i + _NBLK)),
            pl.BlockSpec((EMB, _VB), lambda i: (0, i + 2 * _NBLK)),
            pl.BlockSpec((EMB, _VB), lambda i: (0, jnp.minimum(i + 3 * _NBLK, _LASTBLK))),
        ],
        out_specs=pl.BlockSpec((_VB, 2 * EMB), lambda i: (i, 0)),
        out_shape=jax.ShapeDtypeStruct((_S, 2 * EMB), jnp.int32),
    )(tablet, tablet, tablet, tablet)


def _sc_gather_body(packed_hbm, idx_hbm, out_hbm, idx_v, rows_v, sem):
    wid = lax.axis_index("s") * _NC + lax.axis_index("c")
    base = wid * _B_PER_W
    pltpu.sync_copy(idx_hbm.at[pl.ds(wid * (_B_PER_W // _CHUNK), _B_PER_W // _CHUNK)], idx_v)
    for c in range(_N_BUFS):
        copies = [
            pltpu.async_copy(
                packed_hbm.at[idx_v.at[c * (_ROWS_PER_BUF // _CHUNK) + j]],
                rows_v.at[pl.ds(j * _CHUNK, _CHUNK)],
                sem,
            )
            for j in range(_ROWS_PER_BUF // _CHUNK)
        ]
        for cp in copies:
            cp.wait()
        pltpu.sync_copy(rows_v, out_hbm.at[pl.ds(base + c * _ROWS_PER_BUF, _ROWS_PER_BUF)])


def _sc_gather(packed, idx2d):
    mesh = plsc.VectorSubcoreMesh(core_axis_name="c", subcore_axis_name="s")
    return pl.kernel(
        _sc_gather_body,
        out_type=jax.ShapeDtypeStruct((_B_TOTAL, 2 * EMB), jnp.int32),
        mesh=mesh,
        scratch_types=[
            pltpu.VMEM((_B_PER_W // _CHUNK, _CHUNK), jnp.int32),
            pltpu.VMEM((_ROWS_PER_BUF, 2 * EMB), jnp.int32),
            pltpu.SemaphoreType.DMA,
        ],
        compiler_params=pltpu.CompilerParams(use_tc_tiling_on_sc=True),
    )(packed, idx2d)


_BM = 4096


def _unpack_select(x, q):
    xu = lax.bitcast_convert_type(x, jnp.uint32)
    xh = jnp.where(q >= 2, xu[:, EMB:], xu[:, :EMB])
    lo_f = lax.bitcast_convert_type(xh << 16, jnp.float32)
    hi_f = lax.bitcast_convert_type(xh & jnp.uint32(0xFFFF0000), jnp.float32)
    return jnp.where((q & 1) == 1, hi_f, lo_f)


def _mlp_body(x1_ref, x2_ref, q1_ref, q2_ref, w1_ref, b1_ref, w2t_ref, b2_ref, out_ref):
    w1 = w1_ref[...]
    h_emb = _unpack_select(x1_ref[...], q1_ref[...])
    t_emb = _unpack_select(x2_ref[...], q2_ref[...])
    h = jnp.dot(h_emb, w1[:EMB], preferred_element_type=jnp.float32)
    h = h + jnp.dot(t_emb, w1[EMB:], preferred_element_type=jnp.float32)
    h = jnp.maximum(h + b1_ref[...], 0.0)
    out = jnp.sum(h * w2t_ref[...], axis=1, keepdims=True) + b2_ref[...]
    out_ref[...] = out


def _tc_mlp(embg, quarter, W1, b1, W2, b2):
    nblk = BATCH // _BM
    return pl.pallas_call(
        _mlp_body,
        grid=(nblk,),
        in_specs=[
            pl.BlockSpec((_BM, 2 * EMB), lambda i: (i, 0)),
            pl.BlockSpec((_BM, 2 * EMB), lambda i: (i + nblk, 0)),
            pl.BlockSpec((_BM, 1), lambda i: (i, 0)),
            pl.BlockSpec((_BM, 1), lambda i: (i + nblk, 0)),
            pl.BlockSpec((2 * EMB, EMB), lambda i: (0, 0)),
            pl.BlockSpec((1, EMB), lambda i: (0, 0)),
            pl.BlockSpec((1, EMB), lambda i: (0, 0)),
            pl.BlockSpec((1, 1), lambda i: (0, 0)),
        ],
        out_specs=pl.BlockSpec((_BM, 1), lambda i: (i, 0)),
        out_shape=jax.ShapeDtypeStruct((BATCH, 1), jnp.float32),
    )(embg, embg, quarter, quarter, W1, b1.reshape(1, EMB), W2.reshape(1, EMB), b2.reshape(1, 1))


def kernel(head, tail, table, W1, b1, W2, b2):
    idx = jnp.concatenate([head, tail]).astype(jnp.int32)
    q = idx // _S
    g = idx - q * _S
    g2d = g.reshape(_B_TOTAL // _CHUNK, _CHUNK)
    packed = _tc_repack(table.T)
    embg = _sc_gather(packed, g2d)
    return _tc_mlp(embg, q.reshape(_B_TOTAL, 1), W1, b1, W2, b2)
